# baseline (speedup 1.0000x reference)
import functools
import math

import jax
import jax.numpy as jnp
from jax.experimental import pallas as pl
from jax.experimental.pallas import tpu as pltpu

_COMPUTE_DTYPE = jnp.bfloat16


def _branch2_kernel(x_ref, w1t_ref, b1_ref, wd_ref, bd_ref,
                    w3t_ref, b3_ref, out_ref, *, H, W):
    HW, NB, Cb = x_ref.shape
    M = HW * NB

    x2 = x_ref[...].reshape(M, Cb)

    t = jnp.dot(x2, w1t_ref[...], preferred_element_type=jnp.float32)
    t = jnp.maximum(t + b1_ref[...], 0.0)

    t4 = t.reshape(H, W, NB, Cb)
    zw = jnp.zeros((H, 1, NB, Cb), jnp.float32)
    tw = jnp.concatenate([zw, t4, zw], axis=1)
    zh = jnp.zeros((1, W + 2, NB, Cb), jnp.float32)
    tp = jnp.concatenate([zh, tw, zh], axis=0)

    wd = wd_ref[...]
    d = None
    for a in range(3):
        for b in range(3):
            term = tp[a:a + H, b:b + W] * wd[3 * a + b].reshape(1, 1, 1, Cb)
            d = term if d is None else d + term
    d = (d + bd_ref[...].reshape(1, 1, 1, Cb)).reshape(M, Cb)

    zo = (jnp.dot(d.astype(_COMPUTE_DTYPE), w3t_ref[...],
                  preferred_element_type=jnp.float32) + b3_ref[...])
    zo = jnp.maximum(zo, 0.0)
    out_ref[...] = zo.astype(out_ref.dtype).reshape(HW, NB, Cb)


def _fold(params):
    w1, s1, b1, wdw, s2, b2, w3, s3, b3 = params
    Cb = w1.shape[0]
    w1t = (w1 * s1[:, None]).T.astype(_COMPUTE_DTYPE)
    b1c = b1.reshape(1, Cb).astype(jnp.float32)
    wdf = (wdw * s2[:, None, None]).reshape(Cb, 9).T.astype(jnp.float32)
    bdc = b2.reshape(1, Cb).astype(jnp.float32)
    w3t = (w3 * s3[:, None]).T.astype(_COMPUTE_DTYPE)
    b3c = b3.reshape(1, Cb).astype(jnp.float32)
    return w1t, b1c, wdf, bdc, w3t, b3c


@jax.jit
def kernel(x, w1, s1, b1, wdw, s2, b2, w3, s3, b3):
    N, C, H, W = x.shape
    HW = H * W
    Cb = C // 2

    w1t, b1c, wdf, bdc, w3t, b3c = _fold(
        (w1, s1, b1, wdw, s2, b2, w3, s3, b3))

    NB = math.gcd(N, 8)
    G = N // NB
    x5 = x.reshape(G, NB, C, HW)
    xt = jnp.transpose(x5[:, :, Cb:, :], (0, 3, 1, 2)).astype(_COMPUTE_DTYPE)

    kernel_fn = functools.partial(_branch2_kernel, H=H, W=W)
    const = lambda a: pl.BlockSpec(a.shape, lambda n: (0,) * a.ndim)

    flops = int(N * (4 * Cb * Cb * HW + 24 * Cb * HW))
    bytes_accessed = int(4 * N * Cb * HW // 2)

    zo = pl.pallas_call(
        kernel_fn,
        out_shape=jax.ShapeDtypeStruct((G, HW, NB, Cb), _COMPUTE_DTYPE),
        grid_spec=pltpu.PrefetchScalarGridSpec(
            num_scalar_prefetch=0,
            grid=(G,),
            in_specs=[
                pl.BlockSpec((pl.Squeezed(), HW, NB, Cb),
                             lambda n: (n, 0, 0, 0)),
                const(w1t), const(b1c), const(wdf), const(bdc),
                const(w3t), const(b3c),
            ],
            out_specs=pl.BlockSpec((pl.Squeezed(), HW, NB, Cb),
                                   lambda n: (n, 0, 0, 0)),
        ),
        compiler_params=pltpu.CompilerParams(
            dimension_semantics=("parallel",)),
        cost_estimate=pl.CostEstimate(flops=flops, transcendentals=0,
                                      bytes_accessed=bytes_accessed),
    )(xt, w1t, b1c, wdf, bdc, w3t, b3c)

    zoc = jnp.transpose(zo, (0, 2, 3, 1)).reshape(N, Cb, H, W)
    evn = x[:, :Cb].astype(_COMPUTE_DTYPE)
    return jnp.stack([evn, zoc], axis=2).reshape(N, C, H, W)


# speedup vs baseline: 1.0660x; 1.0660x over previous
import functools
import math

import jax
import jax.numpy as jnp
from jax.experimental import pallas as pl
from jax.experimental.pallas import tpu as pltpu

_COMPUTE_DTYPE = jnp.bfloat16


def _branch2_kernel(x_ref, w1t_ref, b1_ref, wd_ref, bd_ref,
                    w3t_ref, b3_ref, out_ref, *, H, W):
    HW, NB, Cb = x_ref.shape
    M = HW * NB

    x2 = x_ref[...].astype(_COMPUTE_DTYPE).reshape(M, Cb)

    t = jnp.dot(x2, w1t_ref[...], preferred_element_type=jnp.float32)
    t = jnp.maximum(t + b1_ref[...], 0.0)

    t4 = t.reshape(H, W, NB, Cb)
    zw = jnp.zeros((H, 1, NB, Cb), jnp.float32)
    tw = jnp.concatenate([zw, t4, zw], axis=1)
    zh = jnp.zeros((1, W + 2, NB, Cb), jnp.float32)
    tp = jnp.concatenate([zh, tw, zh], axis=0)

    wd = wd_ref[...]
    d = None
    for a in range(3):
        for b in range(3):
            term = tp[a:a + H, b:b + W] * wd[3 * a + b].reshape(1, 1, 1, Cb)
            d = term if d is None else d + term
    d = (d + bd_ref[...].reshape(1, 1, 1, Cb)).reshape(M, Cb)

    zo = (jnp.dot(d.astype(_COMPUTE_DTYPE), w3t_ref[...],
                  preferred_element_type=jnp.float32) + b3_ref[...])
    zo = jnp.maximum(zo, 0.0)
    out_ref[...] = zo.astype(out_ref.dtype).reshape(HW, NB, Cb)


def _fold(params):
    w1, s1, b1, wdw, s2, b2, w3, s3, b3 = params
    Cb = w1.shape[0]
    w1t = (w1 * s1[:, None]).T.astype(_COMPUTE_DTYPE)
    b1c = b1.reshape(1, Cb).astype(jnp.float32)
    wdf = (wdw * s2[:, None, None]).reshape(Cb, 9).T.astype(jnp.float32)
    bdc = b2.reshape(1, Cb).astype(jnp.float32)
    w3t = (w3 * s3[:, None]).T.astype(_COMPUTE_DTYPE)
    b3c = b3.reshape(1, Cb).astype(jnp.float32)
    return w1t, b1c, wdf, bdc, w3t, b3c


@jax.jit
def kernel(x, w1, s1, b1, wdw, s2, b2, w3, s3, b3):
    N, C, H, W = x.shape
    HW = H * W
    Cb = C // 2

    w1t, b1c, wdf, bdc, w3t, b3c = _fold(
        (w1, s1, b1, wdw, s2, b2, w3, s3, b3))

    NB = math.gcd(N, 16)
    G = N // NB
    x5 = x.reshape(G, NB, C, HW)
    xt = jnp.transpose(x5[:, :, Cb:, :], (0, 3, 1, 2))

    kernel_fn = functools.partial(_branch2_kernel, H=H, W=W)
    const = lambda a: pl.BlockSpec(a.shape, lambda n: (0,) * a.ndim)

    flops = int(N * (4 * Cb * Cb * HW + 24 * Cb * HW))
    bytes_accessed = int(3 * N * Cb * HW)

    zo = pl.pallas_call(
        kernel_fn,
        out_shape=jax.ShapeDtypeStruct((G, HW, NB, Cb), _COMPUTE_DTYPE),
        grid_spec=pltpu.PrefetchScalarGridSpec(
            num_scalar_prefetch=0,
            grid=(G,),
            in_specs=[
                pl.BlockSpec((pl.Squeezed(), HW, NB, Cb),
                             lambda n: (n, 0, 0, 0)),
                const(w1t), const(b1c), const(wdf), const(bdc),
                const(w3t), const(b3c),
            ],
            out_specs=pl.BlockSpec((pl.Squeezed(), HW, NB, Cb),
                                   lambda n: (n, 0, 0, 0)),
        ),
        compiler_params=pltpu.CompilerParams(
            dimension_semantics=("parallel",)),
        cost_estimate=pl.CostEstimate(flops=flops, transcendentals=0,
                                      bytes_accessed=bytes_accessed),
    )(xt, w1t, b1c, wdf, bdc, w3t, b3c)

    zoc = jnp.transpose(zo, (0, 2, 3, 1)).reshape(N, Cb, H, W)
    evn = x[:, :Cb].astype(_COMPUTE_DTYPE)
    return jnp.stack([evn, zoc], axis=2).reshape(N, C, H, W)


# speedup vs baseline: 1.0863x; 1.0190x over previous
import functools
import math

import jax
import jax.numpy as jnp
from jax.experimental import pallas as pl
from jax.experimental.pallas import tpu as pltpu

_COMPUTE_DTYPE = jnp.bfloat16


def _branch2_kernel(x_ref, w1t_ref, b1_ref, wd_ref, bd_ref,
                    w3t_ref, b3_ref, out_ref, *, H, W):
    HW, NB, Cb = x_ref.shape
    M = HW * NB

    x2 = x_ref[...].astype(_COMPUTE_DTYPE).reshape(M, Cb)

    t = jnp.dot(x2, w1t_ref[...], preferred_element_type=jnp.float32)
    t = jnp.maximum(t + b1_ref[...], 0.0)

    t4 = t.reshape(H, W, NB, Cb)
    zw = jnp.zeros((H, 1, NB, Cb), jnp.float32)
    tw = jnp.concatenate([zw, t4, zw], axis=1)
    zh = jnp.zeros((1, W + 2, NB, Cb), jnp.float32)
    tp = jnp.concatenate([zh, tw, zh], axis=0)

    wd = wd_ref[...]
    d = None
    for a in range(3):
        for b in range(3):
            term = tp[a:a + H, b:b + W] * wd[3 * a + b].reshape(1, 1, 1, Cb)
            d = term if d is None else d + term
    d = (d + bd_ref[...].reshape(1, 1, 1, Cb)).reshape(M, Cb)

    zo = (jnp.dot(d.astype(_COMPUTE_DTYPE), w3t_ref[...],
                  preferred_element_type=jnp.float32) + b3_ref[...])
    zo = jnp.maximum(zo, 0.0)
    out_ref[...] = zo.astype(out_ref.dtype).reshape(HW, NB, Cb)


def _fold(params):
    w1, s1, b1, wdw, s2, b2, w3, s3, b3 = params
    Cb = w1.shape[0]
    w1t = (w1 * s1[:, None]).T.astype(_COMPUTE_DTYPE)
    b1c = b1.reshape(1, Cb).astype(jnp.float32)
    wdf = (wdw * s2[:, None, None]).reshape(Cb, 9).T.astype(jnp.float32)
    bdc = b2.reshape(1, Cb).astype(jnp.float32)
    w3t = (w3 * s3[:, None]).T.astype(_COMPUTE_DTYPE)
    b3c = b3.reshape(1, Cb).astype(jnp.float32)
    return w1t, b1c, wdf, bdc, w3t, b3c


@jax.jit
def kernel(x, w1, s1, b1, wdw, s2, b2, w3, s3, b3):
    N, C, H, W = x.shape
    HW = H * W
    Cb = C // 2

    w1t, b1c, wdf, bdc, w3t, b3c = _fold(
        (w1, s1, b1, wdw, s2, b2, w3, s3, b3))

    NB = math.gcd(N, 8)
    G = N // NB
    x5 = x.reshape(G, NB, C, HW)
    xt = jnp.transpose(x5[:, :, Cb:, :], (0, 3, 1, 2))

    kernel_fn = functools.partial(_branch2_kernel, H=H, W=W)
    const = lambda a: pl.BlockSpec(a.shape, lambda n: (0,) * a.ndim)

    flops = int(N * (4 * Cb * Cb * HW + 24 * Cb * HW))
    bytes_accessed = int(3 * N * Cb * HW)

    zo = pl.pallas_call(
        kernel_fn,
        out_shape=jax.ShapeDtypeStruct((G, HW, NB, Cb), _COMPUTE_DTYPE),
        grid_spec=pltpu.PrefetchScalarGridSpec(
            num_scalar_prefetch=0,
            grid=(G,),
            in_specs=[
                pl.BlockSpec((pl.Squeezed(), HW, NB, Cb),
                             lambda n: (n, 0, 0, 0)),
                const(w1t), const(b1c), const(wdf), const(bdc),
                const(w3t), const(b3c),
            ],
            out_specs=pl.BlockSpec((pl.Squeezed(), HW, NB, Cb),
                                   lambda n: (n, 0, 0, 0)),
        ),
        compiler_params=pltpu.CompilerParams(
            dimension_semantics=("arbitrary",)),
        cost_estimate=pl.CostEstimate(flops=flops, transcendentals=0,
                                      bytes_accessed=bytes_accessed),
    )(xt, w1t, b1c, wdf, bdc, w3t, b3c)

    zoc = jnp.transpose(zo, (0, 2, 3, 1)).reshape(N, Cb, H, W)
    evn = x[:, :Cb].astype(_COMPUTE_DTYPE)
    return jnp.stack([evn, zoc], axis=2).reshape(N, C, H, W)
